# Initial kernel scaffold; baseline (speedup 1.0000x reference)
import jax
import jax.numpy as jnp
from jax.experimental import pallas as pl


def kernel(x, edge_index, W, a_src, a_tgt):
    raise NotImplementedError("write your pallas kernel here")


# speedup vs baseline: 19.2980x; 19.2980x over previous
import functools

import jax
import jax.numpy as jnp
from jax import lax
from jax.experimental import pallas as pl
from jax.experimental.pallas import tpu as pltpu
from jax.experimental.pallas import tpu_sc as plsc

N = 10000
E = 320000
D = 128

NC = 2
NS = 16
L = 16
NW = NC * NS
EPW = E // NW
K = 80
SCK = 5
SCE = SCK * K
NSUP = EPW // SCE
RPT = 624
CB = 16

BN = 2000
BNZ = 1000


def _proj_body(x_ref, w_ref, as_ref, at_ref, h_ref, s_ref, t_ref,
               smax_ref, tmax_ref):
    i = pl.program_id(0)
    xb = x_ref[...]
    hb = lax.dot_general(xb, w_ref[...], (((1,), (1,)), ((), ())),
                         preferred_element_type=jnp.float32)
    h_ref[...] = hb
    sb = lax.dot_general(hb, as_ref[...], (((1,), (0,)), ((), ())),
                         preferred_element_type=jnp.float32)
    tb = lax.dot_general(hb, at_ref[...], (((1,), (0,)), ((), ())),
                         preferred_element_type=jnp.float32)
    s_ref[...] = sb
    t_ref[...] = tb

    @pl.when(i == 0)
    def _():
        smax_ref[...] = jnp.full((1, 1), -jnp.inf, jnp.float32)
        tmax_ref[...] = jnp.full((1, 1), -jnp.inf, jnp.float32)

    smax_ref[...] = jnp.maximum(smax_ref[...], jnp.max(sb))
    tmax_ref[...] = jnp.maximum(tmax_ref[...], jnp.max(tb))


_proj_call = pl.pallas_call(
    _proj_body,
    grid=(N // BN,),
    in_specs=[
        pl.BlockSpec((BN, D), lambda i: (i, 0)),
        pl.BlockSpec((D, D), lambda i: (0, 0)),
        pl.BlockSpec((D, 1), lambda i: (0, 0)),
        pl.BlockSpec((D, 1), lambda i: (0, 0)),
    ],
    out_specs=[
        pl.BlockSpec((BN, D), lambda i: (i, 0)),
        pl.BlockSpec((BN, 1), lambda i: (i, 0)),
        pl.BlockSpec((BN, 1), lambda i: (i, 0)),
        pl.BlockSpec((1, 1), lambda i: (0, 0)),
        pl.BlockSpec((1, 1), lambda i: (0, 0)),
    ],
    out_shape=[
        jax.ShapeDtypeStruct((N, D), jnp.float32),
        jax.ShapeDtypeStruct((N, 1), jnp.float32),
        jax.ShapeDtypeStruct((N, 1), jnp.float32),
        jax.ShapeDtypeStruct((1, 1), jnp.float32),
        jax.ShapeDtypeStruct((1, 1), jnp.float32),
    ],
)


def _sc_body(h_hbm, s_hbm, t_hbm, src_hbm, tgt_hbm, c_hbm,
             part_hbm, zp_hbm,
             s_v, t_v, z_v, w_v, src_v, tgt_v, c_v, rows_v,
             acc_sh, sem):
    cid = lax.axis_index("c")
    sid = lax.axis_index("s")
    wid = cid * NS + sid

    pltpu.sync_copy(s_hbm, s_v)
    pltpu.sync_copy(t_hbm, t_v)
    pltpu.sync_copy(c_hbm, c_v)

    zeros = jnp.zeros((L,), jnp.float32)

    def _zz(i, carry):
        z_v[pl.ds(i * L, L)] = zeros
        return carry
    lax.fori_loop(0, N // L, _zz, 0)

    def _zc(i, carry):
        rows_v[i // (D // L), pl.ds((i % (D // L)) * L, L)] = zeros
        return carry
    lax.fori_loop(0, CB * D // L, _zc, 0)

    nch = jnp.where(sid == NS - 1, (N - (NS - 1) * RPT) // CB, RPT // CB)

    def _za(m, carry):
        pltpu.sync_copy(rows_v.at[pl.ds(0, CB)],
                        acc_sh.at[pl.ds(sid * RPT + m * CB, CB)])
        return carry
    lax.fori_loop(0, nch, _za, 0)

    plsc.subcore_barrier()

    cvec = c_v[...]

    def _super(g, carry):
        pltpu.sync_copy(src_hbm.at[wid, g], src_v)
        pltpu.sync_copy(tgt_hbm.at[wid, g], tgt_v)

        def _pa(j, carry1):
            def _pa_inner(k, carry2):
                sl = pl.ds(k * L, L)
                si = src_v[j, sl]
                ti = tgt_v[j, sl]
                sv = plsc.load_gather(s_v, [si])
                tv = plsc.load_gather(t_v, [ti])
                e = sv + tv
                e = jnp.where(e > 0, e, 0.2 * e)
                w = jnp.exp(e - cvec)
                w_v[j, sl] = w
                plsc.addupdate_scatter(z_v, [ti], w)
                return carry2
            return lax.fori_loop(0, K // L, _pa_inner, carry1)
        lax.fori_loop(0, SCK, _pa, 0)

        def _pb(j, carry1):
            pltpu.async_copy(h_hbm.at[src_v.at[j]], rows_v, sem).wait()

            def _scale(q, carry2):
                wv = w_v[j, pl.ds(q * L, L)]
                for r16 in range(L):
                    w = wv[r16]
                    r = q * L + r16
                    for c in range(D // L):
                        sl = pl.ds(c * L, L)
                        rows_v[r, sl] = rows_v[r, sl] * w
                return carry2
            lax.fori_loop(0, K // L, _scale, 0)
            pltpu.sync_copy(rows_v, acc_sh.at[tgt_v.at[j]], add=True)
            return carry1
        lax.fori_loop(0, SCK, _pb, 0)
        return carry
    lax.fori_loop(0, NSUP, _super, 0)

    plsc.subcore_barrier()

    def _out(m, carry):
        base = sid * RPT + m * CB
        pltpu.sync_copy(acc_sh.at[pl.ds(base, CB)], rows_v.at[pl.ds(0, CB)])
        pltpu.sync_copy(rows_v.at[pl.ds(0, CB)], part_hbm.at[cid, pl.ds(base, CB)])
        return carry
    lax.fori_loop(0, nch, _out, 0)

    def _zout(m, carry):
        pltpu.sync_copy(z_v.at[pl.ds(m * BNZ, BNZ)], zp_hbm.at[m, wid, 0])
        return carry
    lax.fori_loop(0, N // BNZ, _zout, 0)


@functools.cache
def _make_sc_call():
  return pl.kernel(
    _sc_body,
    out_type=[
        jax.ShapeDtypeStruct((NC, N, D), jnp.float32),
        jax.ShapeDtypeStruct((N // BNZ, NW, 1, BNZ), jnp.float32),
    ],
    mesh=plsc.VectorSubcoreMesh(core_axis_name="c", subcore_axis_name="s",
                                num_cores=NC, num_subcores=NS),
    compiler_params=pltpu.CompilerParams(needs_layout_passes=False),
    scratch_types=[
        pltpu.VMEM((N,), jnp.float32),
        pltpu.VMEM((N,), jnp.float32),
        pltpu.VMEM((N,), jnp.float32),
        pltpu.VMEM((SCK, K), jnp.float32),
        pltpu.VMEM((SCK, K), jnp.int32),
        pltpu.VMEM((SCK, K), jnp.int32),
        pltpu.VMEM((L,), jnp.float32),
        pltpu.VMEM((K, D), jnp.float32),
        pltpu.VMEM_SHARED((N, D), jnp.float32),
        pltpu.SemaphoreType.DMA,
    ],
  )


def _norm_body(part_ref, zp_ref, eye_ref, out_ref):
    p = part_ref[...]
    zl = jnp.sum(zp_ref[...], axis=(0, 1, 2)).reshape(1, BNZ)
    recip = 1.0 / (zl + 1e-10)
    diag = eye_ref[...] * recip
    psum = p[0] + p[1]
    out_ref[...] = lax.dot_general(diag, psum, (((1,), (0,)), ((), ())),
                                   preferred_element_type=jnp.float32)


_norm_call = pl.pallas_call(
    _norm_body,
    grid=(N // BNZ,),
    in_specs=[
        pl.BlockSpec((NC, BNZ, D), lambda i: (0, i, 0)),
        pl.BlockSpec((1, NW, 1, BNZ), lambda i: (i, 0, 0, 0)),
        pl.BlockSpec((BNZ, BNZ), lambda i: (0, 0)),
    ],
    out_specs=pl.BlockSpec((BNZ, D), lambda i: (i, 0)),
    out_shape=jax.ShapeDtypeStruct((N, D), jnp.float32),
)


def kernel(x, edge_index, W, a_src, a_tgt):
    h, s, t, smax, tmax = _proj_call(x, W, a_src, a_tgt)
    c16 = jnp.broadcast_to(smax[0, 0] + tmax[0, 0], (L,))
    src_r = edge_index[0].reshape(NW, NSUP, SCK, K)
    tgt_r = edge_index[1].reshape(NW, NSUP, SCK, K)
    part, zp = _make_sc_call()(h, s.reshape(N), t.reshape(N), src_r, tgt_r, c16)
    return _norm_call(part, zp, jnp.eye(BNZ, dtype=jnp.float32))
